# baseline (speedup 1.0000x reference)
import jax
import jax.numpy as jnp
from jax.experimental import pallas as pl
from jax.experimental.pallas import tpu as pltpu

_NLAT_IN = 128
_NLON_IN = 256
_NLAT_OUT = 64
_NLON_OUT = 128
_K = 3
_BAND = 4


def _body(offs_ref, dla_ref, r_ref, m0_ref, k_ref, vals_ref, x_ref, out_ref):
    t = pl.program_id(0)
    out_ref[...] = jnp.zeros_like(out_ref)
    e0 = offs_ref[t]
    e1 = offs_ref[t + 1]

    def step(e, carry):
        row = x_ref[0, dla_ref[e], r_ref[e], :, :]
        win = pltpu.roll(row, m0_ref[e], axis=1)
        k = k_ref[e]
        out_ref[0, k, :, :] = out_ref[0, k, :, :] + vals_ref[e] * win
        return carry

    jax.lax.fori_loop(e0, e1, step, 0)


def kernel(x, psi_vals, psi_idx):
    B, C = x.shape[0], x.shape[1]
    BC = B * C

    xbc = x.reshape(BC, _NLAT_IN, _NLON_IN).transpose(1, 2, 0)
    xR = xbc[:, ::-1, :]
    xrev = jnp.stack([xR[:, 1::2, :], xR[:, 0::2, :]], axis=1).transpose(0, 1, 3, 2)
    la0s = jnp.clip(2 * jnp.arange(_NLAT_OUT) - 1, 0, _NLAT_IN - _BAND)
    xbands = xrev[la0s[:, None] + jnp.arange(_BAND)[None, :]]

    kk = psi_idx[0].astype(jnp.int32)
    tt = psi_idx[1].astype(jnp.int32)
    cc = psi_idx[2].astype(jnp.int32)
    la = cc // _NLON_IN
    lo = cc - la * _NLON_IN
    r = lo & 1
    m0 = lo >> 1
    la0 = jnp.clip(2 * tt - 1, 0, _NLAT_IN - _BAND)
    dla = la - la0
    offs = jnp.searchsorted(
        tt, jnp.arange(_NLAT_OUT + 1, dtype=jnp.int32), side='left'
    ).astype(jnp.int32)

    grid_spec = pltpu.PrefetchScalarGridSpec(
        num_scalar_prefetch=6,
        grid=(_NLAT_OUT,),
        in_specs=[
            pl.BlockSpec(
                (1, _BAND, 2, BC, _NLON_OUT),
                lambda t, *_: (t, 0, 0, 0, 0),
            )
        ],
        out_specs=pl.BlockSpec(
            (1, _K, BC, _NLON_OUT), lambda t, *_: (t, 0, 0, 0)
        ),
    )
    out = pl.pallas_call(
        _body,
        grid_spec=grid_spec,
        out_shape=jax.ShapeDtypeStruct((_NLAT_OUT, _K, BC, _NLON_OUT), jnp.float32),
    )(offs, dla, r, m0, kk, psi_vals, xbands)

    return out.transpose(2, 1, 0, 3).reshape(B, C, _K, _NLAT_OUT, _NLON_OUT)


# speedup vs baseline: 4.3236x; 4.3236x over previous
import jax
import jax.numpy as jnp
from jax.experimental import pallas as pl
from jax.experimental.pallas import tpu as pltpu

_NLAT_IN = 128
_NLON_IN = 256
_NLAT_OUT = 64
_NLON_OUT = 128
_K = 3


def _body(offs_ref, la_ref, r_ref, shift_ref, k_ref, vals_ref, x_ref, out_ref):
    t = pl.program_id(0)
    out_ref[...] = jnp.zeros_like(out_ref)
    e0 = offs_ref[t]
    e1 = offs_ref[t + 1]

    def step(e, carry):
        row = x_ref[la_ref[e], r_ref[e], :, :]
        win = pltpu.roll(row, shift_ref[e], axis=0)
        k = k_ref[e]
        out_ref[0, k, :, :] = out_ref[0, k, :, :] + vals_ref[e] * win
        return carry

    jax.lax.fori_loop(e0, e1, step, 0)


def kernel(x, psi_vals, psi_idx):
    B, C = x.shape[0], x.shape[1]
    BC = B * C

    xpar = x.reshape(BC, _NLAT_IN, _NLON_OUT, 2).transpose(1, 3, 2, 0)

    kk = psi_idx[0].astype(jnp.int32)
    tt = psi_idx[1].astype(jnp.int32)
    cc = psi_idx[2].astype(jnp.int32)
    la = cc // _NLON_IN
    lo = cc - la * _NLON_IN
    r = lo & 1
    m0 = lo >> 1
    shift = (_NLON_OUT - m0) & (_NLON_OUT - 1)
    offs = jnp.searchsorted(
        tt, jnp.arange(_NLAT_OUT + 1, dtype=jnp.int32), side='left'
    ).astype(jnp.int32)

    grid_spec = pltpu.PrefetchScalarGridSpec(
        num_scalar_prefetch=6,
        grid=(_NLAT_OUT,),
        in_specs=[
            pl.BlockSpec(
                (_NLAT_IN, 2, _NLON_OUT, BC),
                lambda t, *_: (0, 0, 0, 0),
            )
        ],
        out_specs=pl.BlockSpec(
            (1, _K, _NLON_OUT, BC), lambda t, *_: (t, 0, 0, 0)
        ),
    )
    out = pl.pallas_call(
        _body,
        grid_spec=grid_spec,
        out_shape=jax.ShapeDtypeStruct((_NLAT_OUT, _K, _NLON_OUT, BC), jnp.float32),
    )(offs, la, r, shift, kk, psi_vals, xpar)

    return out[:, :, ::-1, :].transpose(3, 1, 0, 2).reshape(
        B, C, _K, _NLAT_OUT, _NLON_OUT
    )
